# baseline (speedup 1.0000x reference)
import functools

import jax
import jax.numpy as jnp
from jax import lax
from jax.experimental import pallas as pl
from jax.experimental.pallas import tpu as pltpu
from jax.experimental.pallas import tpu_sc as plsc

_ALPHA = 0.1
_HOP = 2
_NLAYERS = 4

_NC = 2
_NS = 16
_NTILES = _NC * _NS
_CH = 128
_NBUF = 8
_LEAD = 4
_GRP = _NBUF * _CH
_SCAN = 2048


def _sc_mesh():
    return plsc.VectorSubcoreMesh(
        core_axis_name="c", subcore_axis_name="s",
        num_cores=_NC, num_subcores=_NS)


@functools.lru_cache(maxsize=None)
def _build_part(npad, epad):
    nblk_all = epad // _CH
    nchunks = epad // _SCAN
    scan_rows = _SCAN // _CH
    rpt = npad // _NTILES
    stg = _SCAN + 2 * _CH

    def body(src_hbm, dst_hbm, psrc_hbm, pdl_hbm, cnt_hbm, deg_hbm,
             csrc_v, cdst_v, ssrc_v, sdl_v, hist_v, cnt_v):
        c = lax.axis_index("c")
        s = lax.axis_index("s")
        w = c * _NS + s
        base_w = w * rpt
        iota16 = lax.iota(jnp.int32, 16)
        zero16 = jnp.zeros((16,), jnp.float32)
        one16f = jnp.full((16,), 1.0, jnp.float32)
        dummy_src = jnp.full((16,), npad - 1, jnp.int32)
        dummy_dl = jnp.full((16,), rpt, jnp.int32)

        def zh(i, carry):
            hist_v[pl.ds(i * 16, 16)] = zero16
            return carry

        lax.fori_loop(0, rpt // 16, zh, 0)

        def chunk(ch, carry):
            flushed_b, off_vec = carry
            pltpu.sync_copy(src_hbm.at[pl.ds(ch * scan_rows, scan_rows)],
                            csrc_v)
            pltpu.sync_copy(dst_hbm.at[pl.ds(ch * scan_rows, scan_rows)],
                            cdst_v)

            @plsc.parallel_loop(0, scan_rows, step=1, unroll=2,
                                carry=off_vec)
            def row(r, off_vec):
                for cc in range(_CH // 16):
                    svec = csrc_v[r, pl.ds(cc * 16, 16)]
                    dvec = cdst_v[r, pl.ds(cc * 16, 16)]
                    m = lax.div(dvec, rpt) == w
                    dl = dvec - base_w
                    mi = m.astype(jnp.int32)
                    pos = off_vec + plsc.cumsum(mi) - mi
                    plsc.store_scatter(ssrc_v, [pos], svec, mask=m)
                    plsc.store_scatter(sdl_v, [pos], dl, mask=m)
                    plsc.addupdate_scatter(hist_v, [dl], one16f, mask=m)
                    off_vec = off_vec + plsc.all_reduce_population_count(m)
                return off_vec

            off_vec = row
            off_s = jnp.max(off_vec)
            nfl = lax.div(off_s, _CH)

            def fl(i, carry):
                pltpu.sync_copy(ssrc_v.at[pl.ds(i * _CH, _CH)],
                                psrc_hbm.at[w, flushed_b + i])
                pltpu.sync_copy(sdl_v.at[pl.ds(i * _CH, _CH)],
                                pdl_hbm.at[w, flushed_b + i])
                return carry

            lax.fori_loop(0, nfl, fl, 0)

            @pl.when(nfl > 0)
            def _():
                for j in range(_CH // 16):
                    gidx = nfl * _CH + j * 16 + iota16
                    sv = plsc.load_gather(ssrc_v, [gidx])
                    dv = plsc.load_gather(sdl_v, [gidx])
                    ssrc_v[pl.ds(j * 16, 16)] = sv
                    sdl_v[pl.ds(j * 16, 16)] = dv

            return flushed_b + nfl, off_vec - nfl * _CH

        flushed_b, off_vec = lax.fori_loop(
            0, nchunks, chunk,
            (jnp.int32(0), jnp.zeros((16,), jnp.int32)))

        off_s = jnp.max(off_vec)
        total = flushed_b * _CH + off_s
        total_pad = lax.div(total + (_GRP - 1), _GRP) * _GRP
        for j in range(_GRP // 16 + 1):
            sidx = off_s + j * 16 + iota16
            plsc.store_scatter(ssrc_v, [sidx], dummy_src)
            plsc.store_scatter(sdl_v, [sidx], dummy_dl)
        nfl2 = lax.div(total_pad, _CH) - flushed_b

        def fl2(i, carry):
            pltpu.sync_copy(ssrc_v.at[pl.ds(i * _CH, _CH)],
                            psrc_hbm.at[w, flushed_b + i])
            pltpu.sync_copy(sdl_v.at[pl.ds(i * _CH, _CH)],
                            pdl_hbm.at[w, flushed_b + i])
            return carry

        lax.fori_loop(0, nfl2, fl2, 0)
        cnt_v[pl.ds(0, 16)] = jnp.full((16,), 1, jnp.int32) * lax.div(
            total_pad, _CH)
        pltpu.sync_copy(cnt_v, cnt_hbm.at[w])
        pltpu.sync_copy(hist_v, deg_hbm.at[pl.ds(base_w, rpt)])

    return pl.kernel(
        body,
        out_type=(
            jax.ShapeDtypeStruct((_NTILES, nblk_all, _CH), jnp.int32),
            jax.ShapeDtypeStruct((_NTILES, nblk_all, _CH), jnp.int32),
            jax.ShapeDtypeStruct((_NTILES, 16), jnp.int32),
            jax.ShapeDtypeStruct((npad,), jnp.float32),
        ),
        mesh=_sc_mesh(),
        compiler_params=pltpu.CompilerParams(use_tc_tiling_on_sc=False,
                                             needs_layout_passes=False),
        scratch_types=[
            pltpu.VMEM((scan_rows, _CH), jnp.int32),
            pltpu.VMEM((scan_rows, _CH), jnp.int32),
            pltpu.VMEM((stg,), jnp.int32),
            pltpu.VMEM((stg,), jnp.int32),
            pltpu.VMEM((npad // _NTILES,), jnp.float32),
            pltpu.VMEM((16,), jnp.int32),
        ],
    )


@functools.lru_cache(maxsize=None)
def _build_spmm(npad, epad, nhid, fuse_mid):
    rpt = npad // _NTILES

    def body(g_hbm, psrc_hbm, pdl_hbm, cnt_hbm, deg_hbm, out_hbm,
             sidx_v, didx_v, rows_v, acc_v, gseg_v, dseg_v, cnt_v,
             gsems, isems_s, isems_d):
        c = lax.axis_index("c")
        s = lax.axis_index("s")
        w = c * _NS + s
        zero16 = jnp.zeros((16,), jnp.float32)
        iota16 = lax.iota(jnp.int32, 16)

        def za(i, carry):
            for j in range(nhid // 16):
                acc_v[i, pl.ds(j * 16, 16)] = zero16
            return carry

        lax.fori_loop(0, rpt + 8, za, 0)
        pltpu.sync_copy(cnt_hbm.at[w], cnt_v)
        nblk = jnp.max(cnt_v[pl.ds(0, 16)])
        groups = lax.div(nblk, _NBUF)

        def start_idx(gi):
            pltpu.async_copy(psrc_hbm.at[w, pl.ds(gi * _NBUF, _NBUF)],
                             sidx_v.at[gi % 4], isems_s.at[gi % 4])
            pltpu.async_copy(pdl_hbm.at[w, pl.ds(gi * _NBUF, _NBUF)],
                             didx_v.at[gi % 4], isems_d.at[gi % 4])

        def wait_idx(gi):
            pltpu.make_async_copy(psrc_hbm.at[w, pl.ds(0, _NBUF)],
                                  sidx_v.at[gi % 4],
                                  isems_s.at[gi % 4]).wait()
            pltpu.make_async_copy(pdl_hbm.at[w, pl.ds(0, _NBUF)],
                                  didx_v.at[gi % 4],
                                  isems_d.at[gi % 4]).wait()

        def start_gather(it, b):
            pltpu.async_copy(g_hbm.at[sidx_v.at[lax.div(it, _NBUF) % 4,
                                                lax.rem(it, _NBUF)]],
                             rows_v.at[b], gsems.at[b])

        def wait_gather(b):
            pltpu.make_async_copy(g_hbm.at[pl.ds(0, _CH)], rows_v.at[b],
                                  gsems.at[b]).wait()

        evecs = [iota16 + eg * 16 for eg in range(_CH // 16)]

        def accum(it, b):
            ib = lax.div(it, _NBUF) % 4
            r = lax.rem(it, _NBUF)
            dls = [didx_v[ib, r, pl.ds(eg * 16, 16)]
                   for eg in range(_CH // 16)]

            @plsc.parallel_loop(0, nhid, step=1, unroll=4)
            def _(t):
                tv = jnp.zeros((16,), jnp.int32) + t
                for eg in range(_CH // 16):
                    vals = plsc.load_gather(rows_v.at[b], [evecs[eg], tv])
                    plsc.addupdate_scatter(acc_v, [dls[eg], tv], vals)

        @pl.when(nblk > 0)
        def _():
            start_idx(0)

            @pl.when(groups > 1)
            def _():
                start_idx(1)

            wait_idx(0)
            for k in range(_LEAD):
                start_gather(k, k % _NBUF)

            def grp(gi, carry):
                base_it = gi * _NBUF
                for b in range(_NBUF):
                    it = base_it + b
                    jt = it + _LEAD
                    jb = (b + _LEAD) % _NBUF
                    if b == 0:
                        @pl.when(gi + 2 < groups)
                        def _():
                            start_idx(gi + 2)
                    if b == _LEAD:
                        @pl.when(gi + 1 < groups)
                        def _():
                            wait_idx(gi + 1)

                    @pl.when(jt < nblk)
                    def _():
                        start_gather(jt, jb)

                    wait_gather(b)
                    accum(it, b)
                return carry

            lax.fori_loop(0, groups, grp, 0)

        if fuse_mid:
            pltpu.sync_copy(g_hbm.at[pl.ds(w * rpt, rpt)], gseg_v)
            pltpu.sync_copy(deg_hbm.at[pl.ds(w * rpt, rpt)], dseg_v)

            def fin(r, carry):
                dv = plsc.load_gather(dseg_v, [jnp.full((16,), 0, jnp.int32)
                                               + r])
                inv = 1.0 / (dv + 1.0)
                for j in range(nhid // 16):
                    cs = pl.ds(j * 16, 16)
                    acc_v[r, cs] = (acc_v[r, cs] + gseg_v[r, cs]) * inv
                return carry

            lax.fori_loop(0, rpt, fin, 0)
        pltpu.sync_copy(acc_v.at[pl.ds(0, rpt)],
                        out_hbm.at[pl.ds(w * rpt, rpt)])

    return pl.kernel(
        body,
        out_type=jax.ShapeDtypeStruct((npad, nhid), jnp.float32),
        mesh=_sc_mesh(),
        compiler_params=pltpu.CompilerParams(use_tc_tiling_on_sc=False,
                                             needs_layout_passes=False),
        scratch_types=[
            pltpu.VMEM((4, _NBUF, _CH), jnp.int32),
            pltpu.VMEM((4, _NBUF, _CH), jnp.int32),
            pltpu.VMEM((_NBUF, _CH, nhid), jnp.float32),
            pltpu.VMEM((rpt + 8, nhid), jnp.float32),
            pltpu.VMEM((rpt, nhid), jnp.float32),
            pltpu.VMEM((rpt,), jnp.float32),
            pltpu.VMEM((16,), jnp.int32),
            pltpu.SemaphoreType.DMA((_NBUF,)),
            pltpu.SemaphoreType.DMA((4,)),
            pltpu.SemaphoreType.DMA((4,)),
        ],
    )


def _dinv_from(deg):
    return lax.rsqrt(jnp.maximum(deg + 1.0, 1.0))


@functools.lru_cache(maxsize=None)
def _build_fc0(npad, nfeat, nhid):
    def body(x_ref, w_ref, b_ref, deg_ref, h_ref, g_ref):
        h = jnp.maximum(
            jnp.dot(x_ref[...], w_ref[...],
                    preferred_element_type=jnp.float32) + b_ref[...], 0.0)
        dinv = _dinv_from(deg_ref[...])
        h_ref[...] = h
        g_ref[...] = h * dinv

    return pl.pallas_call(
        body,
        out_shape=[jax.ShapeDtypeStruct((npad, nhid), jnp.float32)] * 2)


@functools.lru_cache(maxsize=None)
def _build_layer(npad, nhid):
    def body(p_ref, g_ref, deg_ref, h0_ref, wc_ref, h_ref, gn_ref):
        dinv = _dinv_from(deg_ref[...])
        hi = (p_ref[...] + g_ref[...]) * dinv
        sup = (1.0 - _ALPHA) * hi + _ALPHA * h0_ref[...]
        h = jnp.maximum(
            jnp.dot(sup, wc_ref[...], preferred_element_type=jnp.float32),
            0.0)
        h_ref[...] = h
        gn_ref[...] = h * dinv

    return pl.pallas_call(
        body,
        out_shape=[jax.ShapeDtypeStruct((npad, nhid), jnp.float32)] * 2)


@functools.lru_cache(maxsize=None)
def _build_final(npad, nhid, ncpad):
    def body(h_ref, wf_ref, bf_ref, o_ref):
        logits = jnp.dot(h_ref[...], wf_ref[...],
                         preferred_element_type=jnp.float32) + bf_ref[...]
        m = jnp.max(logits, axis=1, keepdims=True)
        sh = logits - m
        lse = jnp.log(jnp.sum(jnp.exp(sh), axis=1, keepdims=True))
        o_ref[...] = sh - lse

    return pl.pallas_call(
        body,
        out_shape=jax.ShapeDtypeStruct((npad, ncpad), jnp.float32))


def kernel(x, edge_index, W_fc0, b_fc0, W_conv, W_fc1, b_fc1):
    n, nfeat = x.shape
    nhid = W_fc0.shape[1]
    ncls = W_fc1.shape[1]
    e = edge_index.shape[1]
    npad = -(-n // (_NS * _CH)) * (_NS * _CH)
    epad = -(-e // _SCAN) * _SCAN

    src = edge_index[0]
    dst = edge_index[1]
    if epad != e:
        fill = jnp.full((epad - e,), npad - 1, dtype=jnp.int32)
        src = jnp.concatenate([src, fill])
        dst = jnp.concatenate([dst, fill])
    src = src.reshape(epad // _CH, _CH)
    dst = dst.reshape(epad // _CH, _CH)
    xp = jnp.zeros((npad, nfeat), x.dtype).at[:n].set(x)

    psrc, pdl, cnt, deg = _build_part(npad, epad)(src, dst)
    deg2 = deg.reshape(npad, 1)

    h, g = _build_fc0(npad, nfeat, nhid)(
        xp, W_fc0, b_fc0.reshape(1, nhid), deg2)
    h0 = h
    spmm_mid = _build_spmm(npad, epad, nhid, True)
    spmm_raw = _build_spmm(npad, epad, nhid, False)
    layer = _build_layer(npad, nhid)
    for _ in range(_NLAYERS):
        gi = g
        for _ in range(_HOP - 1):
            gi = spmm_mid(gi, psrc, pdl, cnt, deg)
        p = spmm_raw(gi, psrc, pdl, cnt, deg)
        h, g = layer(p, gi, deg2, h0, W_conv)

    ncpad = -(-ncls // 64) * 64
    Wf = jnp.zeros((nhid, ncpad), W_fc1.dtype).at[:, :ncls].set(W_fc1)
    bf = jnp.full((1, ncpad), -1e30, jnp.float32).at[0, :ncls].set(b_fc1)
    out = _build_final(npad, nhid, ncpad)(h, Wf, bf)
    return out[:n, :ncls]


# speedup vs baseline: 3.9295x; 3.9295x over previous
import functools

import jax
import jax.numpy as jnp
from jax import lax
from jax.experimental import pallas as pl
from jax.experimental.pallas import tpu as pltpu
from jax.experimental.pallas import tpu_sc as plsc

_ALPHA = 0.1
_HOP = 2
_NLAYERS = 4

_NC = 2
_NS = 16
_NTILES = _NC * _NS
_CH = 128


def _sc_mesh():
    return plsc.VectorSubcoreMesh(
        core_axis_name="c", subcore_axis_name="s",
        num_cores=_NC, num_subcores=_NS)


@functools.lru_cache(maxsize=None)
def _build_deg(npad, epad):
    ept = epad // _NTILES
    iters = ept // _CH
    rows_pt = npad // _NS

    def body(dst_hbm, out_hbm, didx_v, ones_v, zbuf_v, acc_sh):
        c = lax.axis_index("c")
        s = lax.axis_index("s")
        w = c * _NS + s
        one16 = jnp.full((16,), 1.0, jnp.float32)
        zero16 = jnp.zeros((16,), jnp.float32)
        for j in range(_CH // 16):
            ones_v[pl.ds(j * 16, 16)] = one16

        def zb(i, carry):
            zbuf_v[pl.ds(i * 16, 16)] = zero16
            return carry

        lax.fori_loop(0, rows_pt // 16, zb, 0)
        r0 = s * rows_pt
        pltpu.sync_copy(zbuf_v, acc_sh.at[pl.ds(r0, rows_pt)])
        pltpu.sync_copy(dst_hbm.at[pl.ds(w * iters, iters)], didx_v)
        plsc.subcore_barrier()

        def step(it, carry):
            pltpu.sync_copy(ones_v, acc_sh.at[didx_v.at[it]], add=True)
            return carry

        lax.fori_loop(0, iters, step, 0)
        plsc.subcore_barrier()
        pltpu.sync_copy(acc_sh.at[pl.ds(r0, rows_pt)],
                        out_hbm.at[c, pl.ds(r0, rows_pt)])

    return pl.kernel(
        body,
        out_type=jax.ShapeDtypeStruct((_NC, npad), jnp.float32),
        mesh=_sc_mesh(),
        compiler_params=pltpu.CompilerParams(use_tc_tiling_on_sc=False),
        scratch_types=[
            pltpu.VMEM((iters, _CH), jnp.int32),
            pltpu.VMEM((_CH,), jnp.float32),
            pltpu.VMEM((rows_pt,), jnp.float32),
            pltpu.VMEM_SHARED((npad,), jnp.float32),
        ],
    )


_NBUF = 8
_LEAD = 4


@functools.lru_cache(maxsize=None)
def _build_spmm(npad, epad, nhid):
    ept = epad // _NTILES
    iters = ept // _CH
    groups = iters // _NBUF
    rows_pt = npad // _NS

    def body(g_hbm, src_hbm, dst_hbm, out_hbm, sidx_v, didx_v, rows_v,
             acc_sh, gsems, ssems):
        c = lax.axis_index("c")
        s = lax.axis_index("s")
        w = c * _NS + s
        zero16 = jnp.zeros((16,), jnp.float32)

        def zb(i, carry):
            for j in range(nhid // 16):
                rows_v[0, i, pl.ds(j * 16, 16)] = zero16
            return carry

        lax.fori_loop(0, _CH, zb, 0)
        r0 = s * rows_pt
        for k in range(rows_pt // _CH):
            pltpu.sync_copy(rows_v.at[0], acc_sh.at[pl.ds(r0 + k * _CH, _CH)])
        row0 = w * iters
        pltpu.sync_copy(src_hbm.at[pl.ds(row0, iters)], sidx_v)
        pltpu.sync_copy(dst_hbm.at[pl.ds(row0, iters)], didx_v)
        plsc.subcore_barrier()

        def start_gather(it, b):
            pltpu.async_copy(g_hbm.at[sidx_v.at[it]], rows_v.at[b],
                             gsems.at[b])

        def wait_gather(b):
            pltpu.make_async_copy(g_hbm.at[pl.ds(0, _CH)], rows_v.at[b],
                                  gsems.at[b]).wait()

        def start_scatter(it, b):
            pltpu.async_copy(rows_v.at[b], acc_sh.at[didx_v.at[it]],
                             ssems.at[b], add=True)

        def wait_scatter(b):
            pltpu.make_async_copy(g_hbm.at[pl.ds(0, _CH)], rows_v.at[b],
                                  ssems.at[b]).wait()

        for k in range(_LEAD):
            start_gather(k, k % _NBUF)

        def grp(gi, carry):
            base_it = gi * _NBUF
            for b in range(_NBUF):
                it = base_it + b
                jt = it + _LEAD
                jb = (b + _LEAD) % _NBUF

                @pl.when(jt < iters)
                def _():
                    @pl.when(it >= _NBUF - _LEAD)
                    def _():
                        wait_scatter(jb)

                    start_gather(jt, jb)

                wait_gather(b)
                start_scatter(it, b)
            return carry

        lax.fori_loop(0, groups, grp, 0)
        for b in range(_NBUF):
            wait_scatter(b)
        plsc.subcore_barrier()
        pltpu.sync_copy(acc_sh.at[pl.ds(r0, rows_pt)],
                        out_hbm.at[c, pl.ds(r0, rows_pt)])

    return pl.kernel(
        body,
        out_type=jax.ShapeDtypeStruct((_NC, npad, nhid), jnp.float32),
        mesh=_sc_mesh(),
        compiler_params=pltpu.CompilerParams(use_tc_tiling_on_sc=False),
        scratch_types=[
            pltpu.VMEM((iters, _CH), jnp.int32),
            pltpu.VMEM((iters, _CH), jnp.int32),
            pltpu.VMEM((_NBUF, _CH, nhid), jnp.float32),
            pltpu.VMEM_SHARED((npad, nhid), jnp.float32),
            pltpu.SemaphoreType.DMA((_NBUF,)),
            pltpu.SemaphoreType.DMA((_NBUF,)),
        ],
    )


def _dinv_from(degT):
    deg = jnp.sum(degT, axis=1, keepdims=True) + 1.0
    return lax.rsqrt(jnp.maximum(deg, 1.0))


@functools.lru_cache(maxsize=None)
def _build_fc0(npad, nfeat, nhid):
    def body(x_ref, w_ref, b_ref, degT_ref, h_ref, g_ref):
        h = jnp.maximum(
            jnp.dot(x_ref[...], w_ref[...],
                    preferred_element_type=jnp.float32) + b_ref[...], 0.0)
        dinv = _dinv_from(degT_ref[...])
        h_ref[...] = h
        g_ref[...] = h * dinv

    return pl.pallas_call(
        body,
        out_shape=[jax.ShapeDtypeStruct((npad, nhid), jnp.float32)] * 2)


@functools.lru_cache(maxsize=None)
def _build_mid(npad, nhid):
    def body(p_ref, g_ref, degT_ref, o_ref):
        deg = jnp.sum(degT_ref[...], axis=1, keepdims=True) + 1.0
        inv = 1.0 / jnp.maximum(deg, 1.0)
        o_ref[...] = (p_ref[0] + p_ref[1] + g_ref[...]) * inv

    return pl.pallas_call(
        body,
        out_shape=jax.ShapeDtypeStruct((npad, nhid), jnp.float32))


@functools.lru_cache(maxsize=None)
def _build_layer(npad, nhid):
    def body(p_ref, g_ref, degT_ref, h0_ref, wc_ref, h_ref, gn_ref):
        dinv = _dinv_from(degT_ref[...])
        hi = (p_ref[0] + p_ref[1] + g_ref[...]) * dinv
        sup = (1.0 - _ALPHA) * hi + _ALPHA * h0_ref[...]
        h = jnp.maximum(
            jnp.dot(sup, wc_ref[...], preferred_element_type=jnp.float32),
            0.0)
        h_ref[...] = h
        gn_ref[...] = h * dinv

    return pl.pallas_call(
        body,
        out_shape=[jax.ShapeDtypeStruct((npad, nhid), jnp.float32)] * 2)


@functools.lru_cache(maxsize=None)
def _build_final(npad, nhid, ncpad):
    def body(h_ref, wf_ref, bf_ref, o_ref):
        logits = jnp.dot(h_ref[...], wf_ref[...],
                         preferred_element_type=jnp.float32) + bf_ref[...]
        m = jnp.max(logits, axis=1, keepdims=True)
        sh = logits - m
        lse = jnp.log(jnp.sum(jnp.exp(sh), axis=1, keepdims=True))
        o_ref[...] = sh - lse

    return pl.pallas_call(
        body,
        out_shape=jax.ShapeDtypeStruct((npad, ncpad), jnp.float32))


def kernel(x, edge_index, W_fc0, b_fc0, W_conv, W_fc1, b_fc1):
    n, nfeat = x.shape
    nhid = W_fc0.shape[1]
    ncls = W_fc1.shape[1]
    e = edge_index.shape[1]
    npad = -(-n // (_NS * _CH)) * (_NS * _CH)
    egrain = _NTILES * _CH * _NBUF
    epad = -(-e // egrain) * egrain

    src = edge_index[0]
    dst = edge_index[1]
    if epad != e:
        fill = jnp.full((epad - e,), npad - 1, dtype=jnp.int32)
        src = jnp.concatenate([src, fill])
        dst = jnp.concatenate([dst, fill])
    src = src.reshape(epad // _CH, _CH)
    dst = dst.reshape(epad // _CH, _CH)
    xp = jnp.zeros((npad, nfeat), x.dtype).at[:n].set(x)

    deg = _build_deg(npad, epad)(dst)
    degT = deg.T

    h, g = _build_fc0(npad, nfeat, nhid)(
        xp, W_fc0, b_fc0.reshape(1, nhid), degT)
    h0 = h
    spmm = _build_spmm(npad, epad, nhid)
    mid = _build_mid(npad, nhid)
    layer = _build_layer(npad, nhid)
    for _ in range(_NLAYERS):
        gi = g
        for _ in range(_HOP - 1):
            p = spmm(gi, src, dst)
            gi = mid(p, gi, degT)
        p = spmm(gi, src, dst)
        h, g = layer(p, gi, degT, h0, W_conv)

    ncpad = -(-ncls // 64) * 64
    Wf = jnp.zeros((nhid, ncpad), W_fc1.dtype).at[:, :ncls].set(W_fc1)
    bf = jnp.full((1, ncpad), -1e30, jnp.float32).at[0, :ncls].set(b_fc1)
    out = _build_final(npad, nhid, ncpad)(h, Wf, bf)
    return out[:n, :ncls]


# speedup vs baseline: 3.9458x; 1.0042x over previous
import functools

import jax
import jax.numpy as jnp
from jax import lax
from jax.experimental import pallas as pl
from jax.experimental.pallas import tpu as pltpu
from jax.experimental.pallas import tpu_sc as plsc

_ALPHA = 0.1
_HOP = 2
_NLAYERS = 4

_NC = 2
_NS = 16
_NTILES = _NC * _NS
_CH = 128


def _sc_mesh():
    return plsc.VectorSubcoreMesh(
        core_axis_name="c", subcore_axis_name="s",
        num_cores=_NC, num_subcores=_NS)


@functools.lru_cache(maxsize=None)
def _build_deg(npad, epad):
    ept = epad // _NTILES
    iters = ept // _CH
    rows_pt = npad // _NS

    def body(dst_hbm, out_hbm, didx_v, ones_v, zbuf_v, acc_sh):
        c = lax.axis_index("c")
        s = lax.axis_index("s")
        w = c * _NS + s
        one16 = jnp.full((16,), 1.0, jnp.float32)
        zero16 = jnp.zeros((16,), jnp.float32)
        for j in range(_CH // 16):
            ones_v[pl.ds(j * 16, 16)] = one16

        def zb(i, carry):
            zbuf_v[pl.ds(i * 16, 16)] = zero16
            return carry

        lax.fori_loop(0, rows_pt // 16, zb, 0)
        r0 = s * rows_pt
        pltpu.sync_copy(zbuf_v, acc_sh.at[pl.ds(r0, rows_pt)])
        pltpu.sync_copy(dst_hbm.at[pl.ds(w * iters, iters)], didx_v)
        plsc.subcore_barrier()

        def step(it, carry):
            pltpu.sync_copy(ones_v, acc_sh.at[didx_v.at[it]], add=True)
            return carry

        lax.fori_loop(0, iters, step, 0)
        plsc.subcore_barrier()
        pltpu.sync_copy(acc_sh.at[pl.ds(r0, rows_pt)],
                        out_hbm.at[c, pl.ds(r0, rows_pt)])

    return pl.kernel(
        body,
        out_type=jax.ShapeDtypeStruct((_NC, npad), jnp.float32),
        mesh=_sc_mesh(),
        compiler_params=pltpu.CompilerParams(use_tc_tiling_on_sc=False),
        scratch_types=[
            pltpu.VMEM((iters, _CH), jnp.int32),
            pltpu.VMEM((_CH,), jnp.float32),
            pltpu.VMEM((rows_pt,), jnp.float32),
            pltpu.VMEM_SHARED((npad,), jnp.float32),
        ],
    )


_NBUF = 8
_LEAD = 4


@functools.lru_cache(maxsize=None)
def _build_spmm(npad, epad, nhid):
    ept = epad // _NTILES
    iters = ept // _CH
    groups = iters // _NBUF
    rows_pt = npad // _NS

    def body(g_hbm, src_hbm, dst_hbm, out_hbm, sidx_v, didx_v, rows_v,
             acc_sh, gsems, ssems):
        c = lax.axis_index("c")
        s = lax.axis_index("s")
        w = c * _NS + s
        zero16 = jnp.zeros((16,), jnp.float32)

        row0 = w * iters
        pltpu.sync_copy(src_hbm.at[pl.ds(row0, iters)], sidx_v)
        pltpu.sync_copy(dst_hbm.at[pl.ds(row0, iters)], didx_v)
        for k in range(1, _LEAD):
            pltpu.async_copy(g_hbm.at[sidx_v.at[k]], rows_v.at[k % _NBUF],
                             gsems.at[k % _NBUF])

        def zb(i, carry):
            for j in range(nhid // 16):
                rows_v[0, i, pl.ds(j * 16, 16)] = zero16
            return carry

        lax.fori_loop(0, _CH, zb, 0)
        r0 = s * rows_pt
        for k in range(rows_pt // _CH):
            pltpu.sync_copy(rows_v.at[0], acc_sh.at[pl.ds(r0 + k * _CH, _CH)])
        pltpu.async_copy(g_hbm.at[sidx_v.at[0]], rows_v.at[0], gsems.at[0])
        plsc.subcore_barrier()

        def start_gather(it, b):
            pltpu.async_copy(g_hbm.at[sidx_v.at[it]], rows_v.at[b],
                             gsems.at[b])

        def wait_gather(b):
            pltpu.make_async_copy(g_hbm.at[pl.ds(0, _CH)], rows_v.at[b],
                                  gsems.at[b]).wait()

        def start_scatter(it, b):
            pltpu.async_copy(rows_v.at[b], acc_sh.at[didx_v.at[it]],
                             ssems.at[b], add=True)

        def wait_scatter(b):
            pltpu.make_async_copy(g_hbm.at[pl.ds(0, _CH)], rows_v.at[b],
                                  ssems.at[b]).wait()

        def grp(gi, carry):
            base_it = gi * _NBUF
            for b in range(_NBUF):
                it = base_it + b
                jt = it + _LEAD
                jb = (b + _LEAD) % _NBUF

                @pl.when(jt < iters)
                def _():
                    @pl.when(it >= _NBUF - _LEAD)
                    def _():
                        wait_scatter(jb)

                    start_gather(jt, jb)

                wait_gather(b)
                start_scatter(it, b)
            return carry

        lax.fori_loop(0, groups, grp, 0)
        for b in range(_NBUF):
            wait_scatter(b)
        plsc.subcore_barrier()
        pltpu.sync_copy(acc_sh.at[pl.ds(r0, rows_pt)],
                        out_hbm.at[c, pl.ds(r0, rows_pt)])

    return pl.kernel(
        body,
        out_type=jax.ShapeDtypeStruct((_NC, npad, nhid), jnp.float32),
        mesh=_sc_mesh(),
        compiler_params=pltpu.CompilerParams(use_tc_tiling_on_sc=False),
        scratch_types=[
            pltpu.VMEM((iters, _CH), jnp.int32),
            pltpu.VMEM((iters, _CH), jnp.int32),
            pltpu.VMEM((_NBUF, _CH, nhid), jnp.float32),
            pltpu.VMEM_SHARED((npad, nhid), jnp.float32),
            pltpu.SemaphoreType.DMA((_NBUF,)),
            pltpu.SemaphoreType.DMA((_NBUF,)),
        ],
    )


def _dinv_from(degT):
    deg = jnp.sum(degT, axis=1, keepdims=True) + 1.0
    return lax.rsqrt(jnp.maximum(deg, 1.0))


@functools.lru_cache(maxsize=None)
def _build_fc0(npad, nfeat, nhid):
    def body(x_ref, w_ref, b_ref, degT_ref, h_ref, g_ref):
        h = jnp.maximum(
            jnp.dot(x_ref[...], w_ref[...],
                    preferred_element_type=jnp.float32) + b_ref[...], 0.0)
        dinv = _dinv_from(degT_ref[...])
        h_ref[...] = h
        g_ref[...] = h * dinv

    return pl.pallas_call(
        body,
        out_shape=[jax.ShapeDtypeStruct((npad, nhid), jnp.float32)] * 2)


@functools.lru_cache(maxsize=None)
def _build_mid(npad, nhid):
    def body(p_ref, g_ref, degT_ref, o_ref):
        deg = jnp.sum(degT_ref[...], axis=1, keepdims=True) + 1.0
        inv = 1.0 / jnp.maximum(deg, 1.0)
        o_ref[...] = (p_ref[0] + p_ref[1] + g_ref[...]) * inv

    return pl.pallas_call(
        body,
        out_shape=jax.ShapeDtypeStruct((npad, nhid), jnp.float32))


@functools.lru_cache(maxsize=None)
def _build_layer(npad, nhid):
    def body(p_ref, g_ref, degT_ref, h0_ref, wc_ref, h_ref, gn_ref):
        dinv = _dinv_from(degT_ref[...])
        hi = (p_ref[0] + p_ref[1] + g_ref[...]) * dinv
        sup = (1.0 - _ALPHA) * hi + _ALPHA * h0_ref[...]
        h = jnp.maximum(
            jnp.dot(sup, wc_ref[...], preferred_element_type=jnp.float32),
            0.0)
        h_ref[...] = h
        gn_ref[...] = h * dinv

    return pl.pallas_call(
        body,
        out_shape=[jax.ShapeDtypeStruct((npad, nhid), jnp.float32)] * 2)


@functools.lru_cache(maxsize=None)
def _build_final(npad, nhid, ncpad):
    def body(h_ref, wf_ref, bf_ref, o_ref):
        logits = jnp.dot(h_ref[...], wf_ref[...],
                         preferred_element_type=jnp.float32) + bf_ref[...]
        m = jnp.max(logits, axis=1, keepdims=True)
        sh = logits - m
        lse = jnp.log(jnp.sum(jnp.exp(sh), axis=1, keepdims=True))
        o_ref[...] = sh - lse

    return pl.pallas_call(
        body,
        out_shape=jax.ShapeDtypeStruct((npad, ncpad), jnp.float32))


def kernel(x, edge_index, W_fc0, b_fc0, W_conv, W_fc1, b_fc1):
    n, nfeat = x.shape
    nhid = W_fc0.shape[1]
    ncls = W_fc1.shape[1]
    e = edge_index.shape[1]
    npad = -(-n // (_NS * _CH)) * (_NS * _CH)
    egrain = _NTILES * _CH * _NBUF
    epad = -(-e // egrain) * egrain

    src = edge_index[0]
    dst = edge_index[1]
    if epad != e:
        fill = jnp.full((epad - e,), npad - 1, dtype=jnp.int32)
        src = jnp.concatenate([src, fill])
        dst = jnp.concatenate([dst, fill])
    src = src.reshape(epad // _CH, _CH)
    dst = dst.reshape(epad // _CH, _CH)
    xp = jnp.zeros((npad, nfeat), x.dtype).at[:n].set(x)

    deg = _build_deg(npad, epad)(dst)
    degT = deg.T

    h, g = _build_fc0(npad, nfeat, nhid)(
        xp, W_fc0, b_fc0.reshape(1, nhid), degT)
    h0 = h
    spmm = _build_spmm(npad, epad, nhid)
    mid = _build_mid(npad, nhid)
    layer = _build_layer(npad, nhid)
    for _ in range(_NLAYERS):
        gi = g
        for _ in range(_HOP - 1):
            p = spmm(gi, src, dst)
            gi = mid(p, gi, degT)
        p = spmm(gi, src, dst)
        h, g = layer(p, gi, degT, h0, W_conv)

    ncpad = -(-ncls // 64) * 64
    Wf = jnp.zeros((nhid, ncpad), W_fc1.dtype).at[:, :ncls].set(W_fc1)
    bf = jnp.full((1, ncpad), -1e30, jnp.float32).at[0, :ncls].set(b_fc1)
    out = _build_final(npad, nhid, ncpad)(h, Wf, bf)
    return out[:n, :ncls]
